# Initial kernel scaffold; baseline (speedup 1.0000x reference)
import jax
import jax.numpy as jnp
from jax.experimental import pallas as pl


def kernel(input_seq, W0, W1, W2):
    raise NotImplementedError("write your pallas kernel here")


# speedup vs baseline: 1.2700x; 1.2700x over previous
import functools

import jax
import jax.numpy as jnp
from jax import lax
from jax.experimental import pallas as pl
from jax.experimental.pallas import tpu as pltpu
from jax.experimental.pallas import tpu_sc as plsc

_VOCAB = 100000
_DIM = 128
_B, _S = 4, 4096
_NUM_LAYERS = 12

_NTOT = _B * _S
_NC, _NS = 2, 16
_NW = _NC * _NS
_PER_W = _NTOT // _NW
_CHUNK = 128
_ROWS_W = _PER_W // _CHUNK
_NBUF = 4


def _gather3(idx2d, w0, w1, w2):
    mesh = plsc.VectorSubcoreMesh(core_axis_name="c", subcore_axis_name="s")

    @functools.partial(
        pl.kernel,
        mesh=mesh,
        out_type=[jax.ShapeDtypeStruct((_NTOT, _DIM), jnp.float32)] * 3,
        scratch_types=[
            pltpu.VMEM((_ROWS_W, _CHUNK), jnp.int32),
            pltpu.VMEM((_NBUF, _CHUNK, _DIM), jnp.float32),
            pltpu.SemaphoreType.DMA((_NBUF,)),
            pltpu.SemaphoreType.DMA((_NBUF,)),
        ],
    )
    def body(idx_hbm, t0, t1, t2, o0, o1, o2, idx_v, bufs, gsem, wsem):
        wid = lax.axis_index("s") * _NC + lax.axis_index("c")
        pltpu.sync_copy(idx_hbm.at[pl.ds(wid * _ROWS_W, _ROWS_W)], idx_v)

        tabs = (t0, t1, t2)
        outs = (o0, o1, o2)
        tasks = [(t, r) for t in range(3) for r in range(_ROWS_W)]
        n = len(tasks)

        def issue_gather(j):
            t, r = tasks[j]
            slot = j % _NBUF
            return pltpu.async_copy(
                tabs[t].at[idx_v.at[r]], bufs.at[slot], gsem.at[slot])

        gh = [None] * n
        wh = [None] * n
        gh[0] = issue_gather(0)
        for j in range(n):
            slot = j % _NBUF
            if j + 1 < n:
                if j + 1 >= _NBUF:
                    wh[j + 1 - _NBUF].wait()
                gh[j + 1] = issue_gather(j + 1)
            gh[j].wait()
            t, r = tasks[j]
            row0 = wid * _PER_W + r * _CHUNK
            wh[j] = pltpu.async_copy(
                bufs.at[slot], outs[t].at[pl.ds(row0, _CHUNK)], wsem.at[slot])
        for j in range(n - _NBUF, n):
            wh[j].wait()

    return body(idx2d, w0, w1, w2)


def kernel(input_seq, W0, W1, W2):
    idx2d = input_seq.reshape(_NTOT // _CHUNK, _CHUNK)
    o0, o1, o2 = _gather3(idx2d, W0, W1, W2)
    ve = [o.reshape(_B, _S, _DIM) for o in (o0, o1, o2)]
    return tuple(ve[i % 3] for i in range(_NUM_LAYERS))


# speedup vs baseline: 1.6080x; 1.2661x over previous
import functools

import jax
import jax.numpy as jnp
from jax import lax
from jax.experimental import pallas as pl
from jax.experimental.pallas import tpu as pltpu
from jax.experimental.pallas import tpu_sc as plsc

_VOCAB = 100000
_DIM = 128
_B, _S = 4, 4096
_NUM_LAYERS = 12

_NTOT = _B * _S
_NC, _NS = 2, 16
_NW = _NC * _NS
_PER_W = _NTOT // _NW
_CHUNK = 128
_ROWS_W = _PER_W // _CHUNK
_NBUF = 4


def _gather3(idx2d, w0, w1, w2):
    mesh = plsc.VectorSubcoreMesh(core_axis_name="c", subcore_axis_name="s")

    @functools.partial(
        pl.kernel,
        mesh=mesh,
        out_type=[jax.ShapeDtypeStruct((_NTOT, _DIM), jnp.float32)] * _NUM_LAYERS,
        scratch_types=[
            pltpu.VMEM((_ROWS_W, _CHUNK), jnp.int32),
            pltpu.VMEM((_NBUF, _CHUNK, _DIM), jnp.float32),
            pltpu.SemaphoreType.DMA((_NBUF,)),
            pltpu.SemaphoreType.DMA((_NBUF,)),
        ],
    )
    def body(idx_hbm, t0, t1, t2, *rest):
        outs = rest[:_NUM_LAYERS]
        idx_v, bufs, gsem, wsem = rest[_NUM_LAYERS:]
        wid = lax.axis_index("s") * _NC + lax.axis_index("c")
        pltpu.sync_copy(idx_hbm.at[pl.ds(wid * _ROWS_W, _ROWS_W)], idx_v)

        tabs = (t0, t1, t2)
        tasks = [(t, r) for t in range(3) for r in range(_ROWS_W)]
        n = len(tasks)

        def issue_gather(j):
            t, r = tasks[j]
            slot = j % _NBUF
            return pltpu.async_copy(
                tabs[t].at[idx_v.at[r]], bufs.at[slot], gsem.at[slot])

        def issue_writebacks(j):
            t, r = tasks[j]
            slot = j % _NBUF
            row0 = wid * _PER_W + r * _CHUNK
            return [
                pltpu.async_copy(
                    bufs.at[slot], outs[l].at[pl.ds(row0, _CHUNK)],
                    wsem.at[slot])
                for l in range(t, _NUM_LAYERS, 3)
            ]

        gh = [None] * n
        wh = [None] * n
        gh[0] = issue_gather(0)
        for j in range(n):
            if j + 1 < n:
                if j + 1 >= _NBUF:
                    for h in wh[j + 1 - _NBUF]:
                        h.wait()
                gh[j + 1] = issue_gather(j + 1)
            gh[j].wait()
            wh[j] = issue_writebacks(j)
        for j in range(n - _NBUF, n):
            for h in wh[j]:
                h.wait()

    return body(idx2d, w0, w1, w2)


def kernel(input_seq, W0, W1, W2):
    idx2d = input_seq.reshape(_NTOT // _CHUNK, _CHUNK)
    outs = _gather3(idx2d, W0, W1, W2)
    return tuple(o.reshape(_B, _S, _DIM) for o in outs)


# speedup vs baseline: 1.6490x; 1.0255x over previous
import functools

import jax
import jax.numpy as jnp
from jax import lax
from jax.experimental import pallas as pl
from jax.experimental.pallas import tpu as pltpu
from jax.experimental.pallas import tpu_sc as plsc

_VOCAB = 100000
_DIM = 128
_B, _S = 4, 4096
_NUM_LAYERS = 12

_NTOT = _B * _S
_NC, _NS = 2, 16
_NW = _NC * _NS
_PER_W = _NTOT // _NW
_CHUNK = 128
_ROWS_W = _PER_W // _CHUNK
_NBUF = 6
_NGIF = 3


def _gather3(idx2d, w0, w1, w2):
    mesh = plsc.VectorSubcoreMesh(core_axis_name="c", subcore_axis_name="s")

    @functools.partial(
        pl.kernel,
        mesh=mesh,
        out_type=[jax.ShapeDtypeStruct((_NTOT, _DIM), jnp.float32)] * _NUM_LAYERS,
        scratch_types=[
            pltpu.VMEM((_ROWS_W, _CHUNK), jnp.int32),
            pltpu.VMEM((_NBUF, _CHUNK, _DIM), jnp.float32),
            pltpu.SemaphoreType.DMA((_NBUF,)),
            pltpu.SemaphoreType.DMA((_NBUF,)),
        ],
    )
    def body(idx_hbm, t0, t1, t2, *rest):
        outs = rest[:_NUM_LAYERS]
        idx_v, bufs, gsem, wsem = rest[_NUM_LAYERS:]
        wid = lax.axis_index("s") * _NC + lax.axis_index("c")
        pltpu.sync_copy(idx_hbm.at[pl.ds(wid * _ROWS_W, _ROWS_W)], idx_v)

        tabs = (t0, t1, t2)
        tasks = [(t, r) for t in range(3) for r in range(_ROWS_W)]
        n = len(tasks)

        def issue_gather(j):
            t, r = tasks[j]
            slot = j % _NBUF
            return pltpu.async_copy(
                tabs[t].at[idx_v.at[r]], bufs.at[slot], gsem.at[slot])

        def issue_writebacks(j):
            t, r = tasks[j]
            slot = j % _NBUF
            row0 = wid * _PER_W + r * _CHUNK
            return [
                pltpu.async_copy(
                    bufs.at[slot], outs[l].at[pl.ds(row0, _CHUNK)],
                    wsem.at[slot])
                for l in range(t, _NUM_LAYERS, 3)
            ]

        gh = [None] * n
        wh = [None] * n
        for j in range(_NGIF):
            gh[j] = issue_gather(j)
        for j in range(n):
            gh[j].wait()
            wh[j] = issue_writebacks(j)
            nxt = j + _NGIF
            if nxt < n:
                if nxt >= _NBUF:
                    for h in wh[nxt - _NBUF]:
                        h.wait()
                gh[nxt] = issue_gather(nxt)
        for j in range(n - _NBUF, n):
            for h in wh[j]:
                h.wait()

    return body(idx2d, w0, w1, w2)


def kernel(input_seq, W0, W1, W2):
    idx2d = input_seq.reshape(_NTOT // _CHUNK, _CHUNK)
    outs = _gather3(idx2d, W0, W1, W2)
    return tuple(o.reshape(_B, _S, _DIM) for o in outs)
